# baseline (speedup 1.0000x reference)
import functools

import jax
import jax.numpy as jnp
from jax import lax
from jax.experimental import pallas as pl
from jax.experimental.pallas import tpu as pltpu
from jax.experimental.pallas import tpu_sc as plsc

H = 128
B = 128
NCORE = 2
NSUB = 16
NW = NCORE * NSUB

N0, N1, N2 = 50000, 100000, 25000
E0, E1 = 200000, 150000
N0P, N1P, N2P = 53248, 102400, 28672
E0P, E1P = 200704, 151552

_MESH = plsc.VectorSubcoreMesh(core_axis_name="c", subcore_axis_name="s")


def _gsum(table, idxs, ndst_p):
    k = len(idxs)
    rows_w = ndst_p // NW
    nfull = rows_w // 512
    tail = (rows_w % 512) // 128

    GK = 2
    units_w = rows_w // 128
    ngrp = (units_w + GK - 1) // GK

    def units(g):
        return range(g * GK, min((g + 1) * GK, units_w))

    @functools.partial(
        pl.kernel,
        out_type=jax.ShapeDtypeStruct((ndst_p, H), jnp.float32),
        mesh=_MESH,
        scratch_types=[
            pltpu.VMEM((2, k, GK, 128), jnp.int32),
            pltpu.VMEM((2, GK, 128, H), jnp.float32),
            pltpu.SemaphoreType.DMA,
            pltpu.SemaphoreType.DMA,
            pltpu.SemaphoreType.DMA,
            pltpu.SemaphoreType.DMA,
        ],
    )
    def kfn(table_h, *rest):
        idx_hs = rest[:k]
        out_h, idx_v, rows_v, semA, semB, sem2, sem3 = rest[k:]
        cid = lax.axis_index("c")
        sid = lax.axis_index("s")
        wid = sid * NCORE + cid
        base_w = wid * rows_w

        def load_idx(g):
            b = g % 2
            return [pltpu.async_copy(
                        idx_hs[j].at[pl.ds(base_w + u * 128, 128)],
                        idx_v.at[b, j, jj], sem2)
                    for j in range(k) for jj, u in enumerate(units(g))]

        def fire_j(g, j):
            b = g % 2
            sem_ = semA if b == 0 else semB
            return [pltpu.async_copy(table_h.at[idx_v.at[b, j, jj]],
                                     rows_v.at[b, jj], sem_, add=(j > 0))
                    for jj, _ in enumerate(units(g))]

        def fire_wb(g):
            b = g % 2
            return [pltpu.async_copy(
                        rows_v.at[b, jj],
                        out_h.at[pl.ds(base_w + u * 128, 128)], sem3)
                    for jj, u in enumerate(units(g))]

        idx_cur = load_idx(0)
        idx_nxt = load_idx(1) if ngrp > 1 else []
        for cp in idx_cur:
            cp.wait()
        ow = fire_j(0, 0)
        wb_prev, wb_cur = [], []
        for g in range(ngrp):
            for cp in wb_prev:
                cp.wait()
            ow_next = []
            if g + 1 < ngrp:
                for cp in idx_nxt:
                    cp.wait()
                ow_next = fire_j(g + 1, 0)
            for cp in ow:
                cp.wait()
            for j in range(1, k):
                ga = fire_j(g, j)
                for cp in ga:
                    cp.wait()
            wb_prev = wb_cur
            wb_cur = fire_wb(g)
            ow = ow_next
            if g + 1 < ngrp:
                idx_nxt = load_idx(g + 2) if g + 2 < ngrp else []
        for cp in wb_prev:
            cp.wait()
        for cp in wb_cur:
            cp.wait()

    return kfn(table, *idxs)


def _scatter(table, src2d, dst2d, ndst_p, Wc):
    C = H // Wc
    rnds = C // NCORE
    ep = src2d.shape[0] * 128
    per_tile = ep // NSUB
    urows_t = per_tile // 128
    GK = 1 if Wc == 32 else 4
    ngrp = (urows_t + GK - 1) // GK
    rows_tile = ndst_p // NSUB
    zr = rows_tile
    while zr * Wc * 4 > 16384:
        zr //= 2
    nz = rows_tile // zr

    def units(g):
        return range(g * GK, min((g + 1) * GK, urows_t))

    @functools.partial(
        pl.kernel,
        out_type=jax.ShapeDtypeStruct((ndst_p, H), jnp.float32),
        mesh=_MESH,
        compiler_params=pltpu.CompilerParams(use_tc_tiling_on_sc=False),
        scratch_types=[
            pltpu.VMEM_SHARED((ndst_p, Wc), jnp.float32),
            pltpu.VMEM((3, GK, 128), jnp.int32),
            pltpu.VMEM((3, GK, 128), jnp.int32),
            pltpu.VMEM((zr, Wc), jnp.float32),
            pltpu.VMEM((2, GK, 128, Wc), jnp.float32),
            pltpu.SemaphoreType.DMA,
            pltpu.SemaphoreType.DMA,
            pltpu.SemaphoreType.DMA,
            pltpu.SemaphoreType.DMA,
            pltpu.SemaphoreType.DMA,
        ],
    )
    def kfn(table_h, src_h, dst_h, out_h, acc, sidx, didx, zbuf, rows,
            semA, semB, semC, semD, sem2):
        cid = lax.axis_index("c")
        sid = lax.axis_index("s")
        ubase = sid * urows_t

        def zb(i, carry):
            for t in range(Wc // 16):
                zbuf[i, pl.ds(t * 16, 16)] = jnp.zeros((16,), jnp.float32)
            return carry
        lax.fori_loop(0, zr, zb, 0)

        for r in range(rnds):
            zcps = [pltpu.async_copy(
                        zbuf, acc.at[pl.ds(sid * rows_tile + i * zr, zr)],
                        sem2)
                    for i in range(nz)]
            for cp in zcps:
                cp.wait()
            plsc.subcore_barrier()

            col0 = (r * NCORE + cid) * Wc
            chunk_t = r * NCORE + cid

            def load_idx(g):
                b = g % 3
                u0 = g * GK
                nk = len(units(g))
                return [pltpu.async_copy(
                            src_h.at[pl.ds(ubase + u0, nk)],
                            sidx.at[b, pl.ds(0, nk)], sem2),
                        pltpu.async_copy(
                            dst_h.at[pl.ds(ubase + u0, nk)],
                            didx.at[b, pl.ds(0, nk)], sem2)]

            def fire_gathers(g):
                b3 = g % 3
                b = g % 2
                sem_ = semA if b == 0 else semB
                for jj, _ in enumerate(units(g)):
                    for t in range(8):
                        v = sidx[b3, jj, pl.ds(t * 16, 16)]
                        sidx[b3, jj, pl.ds(t * 16, 16)] = v * C + chunk_t
                return [pltpu.async_copy(
                            table_h.at[sidx.at[b3, jj]],
                            rows.at[b, jj], sem_)
                        for jj, _ in enumerate(units(g))]

            def fire_scatter(g):
                b3 = g % 3
                b = g % 2
                sem_ = semC if b == 0 else semD
                return [pltpu.async_copy(rows.at[b, jj],
                                         acc.at[didx.at[b3, jj]], sem_,
                                         add=True)
                        for jj, _ in enumerate(units(g))]

            for cp in load_idx(0):
                cp.wait()
            gat = fire_gathers(0)
            idx_nxt = load_idx(1) if ngrp > 1 else []
            sca_prev = []
            for g in range(ngrp):
                for cp in idx_nxt:
                    cp.wait()
                for cp in sca_prev:
                    cp.wait()
                gat_next = fire_gathers(g + 1) if g + 1 < ngrp else []
                idx_nxt = load_idx(g + 2) if g + 2 < ngrp else []
                for cp in gat:
                    cp.wait()
                sca_prev = fire_scatter(g)
                gat = gat_next
            for cp in sca_prev:
                cp.wait()
            plsc.subcore_barrier()

            ob = sid * rows_tile
            pltpu.sync_copy(acc.at[pl.ds(ob, rows_tile)],
                            out_h.at[pl.ds(ob, rows_tile), pl.ds(col0, Wc)])
            plsc.subcore_barrier()

    return kfn(table, src2d, dst2d)


def _atom_encode(featp, emb):
    n0p = featp.shape[0]
    R = 512

    def body(feat_ref, emb_ref, out_ref):
        acc = jnp.zeros((R, H), jnp.float32)
        for c in range(9):
            col = feat_ref[:, c:c + 1]
            oh = (col == lax.broadcasted_iota(jnp.int32, (R, 64), 1)
                  ).astype(jnp.float32)
            acc = acc + jnp.dot(oh, emb_ref[c],
                                preferred_element_type=jnp.float32, precision=lax.Precision.HIGHEST)
        out_ref[...] = acc

    return pl.pallas_call(
        body,
        grid=(n0p // R,),
        in_specs=[pl.BlockSpec((R, 16), lambda i: (i, 0)),
                  pl.BlockSpec((9, 64, H), lambda i: (0, 0, 0))],
        out_specs=pl.BlockSpec((R, H), lambda i: (i, 0)),
        out_shape=jax.ShapeDtypeStruct((n0p, H), jnp.float32),
    )(featp, emb)


def _mlp(x, aggu, aggf, W1u, b1u, W1f, b1f, W2cat, b2cat, Wcm, bcm):
    np_ = x.shape[0]
    R = 512
    has_u = aggu is not None
    has_f = aggf is not None

    def body(*refs):
        i = 0
        x_ref = refs[i]; i += 1
        if has_u:
            aggu_ref = refs[i]; i += 1
        if has_f:
            aggf_ref = refs[i]; i += 1
        (W1u_ref, b1u_ref, W1f_ref, b1f_ref, W2_ref, b2_ref, Wcm_ref,
         bcm_ref) = refs[i:i + 8]
        out_refs = refs[i + 8:]
        xv = x_ref[...]
        if has_u:
            hu = xv + aggu_ref[...].astype(jnp.float32)
        else:
            hu = xv
        if has_f:
            hf = xv + aggf_ref[...]
        else:
            hf = xv
        au = jnp.maximum(jnp.dot(hu, W1u_ref[...],
                                 preferred_element_type=jnp.float32, precision=lax.Precision.HIGHEST)
                         + b1u_ref[...], 0.0)
        af = jnp.maximum(jnp.dot(hf, W1f_ref[...],
                                 preferred_element_type=jnp.float32, precision=lax.Precision.HIGHEST)
                         + b1f_ref[...], 0.0)
        cat = jnp.concatenate([au, af], axis=1)
        s = jnp.dot(cat, W2_ref[...], preferred_element_type=jnp.float32, precision=lax.Precision.HIGHEST) \
            + b2_ref[...]
        res = jnp.maximum(
            jnp.dot(s, Wcm_ref[...], preferred_element_type=jnp.float32, precision=lax.Precision.HIGHEST)
            + bcm_ref[...], 0.0)
        out_refs[0][...] = res

    in_specs = [pl.BlockSpec((R, H), lambda i: (i, 0))]
    args = [x]
    if has_u:
        in_specs.append(pl.BlockSpec((R, H), lambda i: (i, 0)))
        args.append(aggu)
    if has_f:
        in_specs.append(pl.BlockSpec((R, H), lambda i: (i, 0)))
        args.append(aggf)
    wspecs = [
        pl.BlockSpec((H, H), lambda i: (0, 0)),
        pl.BlockSpec((1, H), lambda i: (0, 0)),
        pl.BlockSpec((H, H), lambda i: (0, 0)),
        pl.BlockSpec((1, H), lambda i: (0, 0)),
        pl.BlockSpec((2 * H, H), lambda i: (0, 0)),
        pl.BlockSpec((1, H), lambda i: (0, 0)),
        pl.BlockSpec((H, H), lambda i: (0, 0)),
        pl.BlockSpec((1, H), lambda i: (0, 0)),
    ]
    out_specs = [pl.BlockSpec((R, H), lambda i: (i, 0))]
    out_shape = [jax.ShapeDtypeStruct((np_, H), jnp.float32)]
    res = pl.pallas_call(
        body,
        grid=(np_ // R,),
        in_specs=in_specs + wspecs,
        out_specs=out_specs,
        out_shape=out_shape,
    )(*args, W1u, b1u, W1f, b1f, W2cat, b2cat, Wcm, bcm)
    return res[0]


def _cast16(x):
    np_ = x.shape[0]
    R = 1024

    def body(x_ref, o_ref):
        o_ref[...] = x_ref[...].astype(jnp.bfloat16)

    return pl.pallas_call(
        body,
        grid=(np_ // R,),
        in_specs=[pl.BlockSpec((R, H), lambda i: (i, 0))],
        out_specs=pl.BlockSpec((R, H), lambda i: (i, 0)),
        out_shape=jax.ShapeDtypeStruct((np_, H), jnp.bfloat16),
    )(x)


def _pool(x, batch2d):
    np_ = x.shape[0]
    R = 512

    def body(x_ref, b_ref, out_ref):
        i = pl.program_id(0)
        oh = (b_ref[...] == lax.broadcasted_iota(jnp.int32, (R, B), 1)
              ).astype(jnp.float32)
        p = lax.dot_general(oh, x_ref[...], (((0,), (0,)), ((), ())),
                            preferred_element_type=jnp.float32, precision=lax.Precision.HIGHEST)

        @pl.when(i == 0)
        def _():
            out_ref[...] = p

        @pl.when(i > 0)
        def _():
            out_ref[...] += p

    return pl.pallas_call(
        body,
        grid=(np_ // R,),
        in_specs=[pl.BlockSpec((R, H), lambda i: (i, 0)),
                  pl.BlockSpec((R, 1), lambda i: (i, 0))],
        out_specs=pl.BlockSpec((B, H), lambda i: (0, 0)),
        out_shape=jax.ShapeDtypeStruct((B, H), jnp.float32),
    )(x, batch2d)


def _final(p0, p1, p2, W1, b1, W2p, b2p):
    def body(p0_ref, p1_ref, p2_ref, W1_ref, b1_ref, W2_ref, b2_ref, out_ref):
        h = jnp.zeros((B, 2 * H), jnp.float32)
        for d, pr in enumerate((p0_ref, p1_ref, p2_ref)):
            h = h + jnp.maximum(
                jnp.dot(pr[...], W1_ref[d], preferred_element_type=jnp.float32, precision=lax.Precision.HIGHEST)
                + b1_ref[d], 0.0)
        out_ref[...] = jnp.dot(h, W2_ref[...],
                               preferred_element_type=jnp.float32, precision=lax.Precision.HIGHEST) + b2_ref[...]

    return pl.pallas_call(
        body,
        in_specs=[pl.BlockSpec((B, H), lambda: (0, 0)),
                  pl.BlockSpec((B, H), lambda: (0, 0)),
                  pl.BlockSpec((B, H), lambda: (0, 0)),
                  pl.BlockSpec((3, H, 2 * H), lambda: (0, 0, 0)),
                  pl.BlockSpec((3, 1, 2 * H), lambda: (0, 0, 0)),
                  pl.BlockSpec((2 * H, H), lambda: (0, 0)),
                  pl.BlockSpec((1, H), lambda: (0, 0))],
        out_specs=pl.BlockSpec((B, H), lambda: (0, 0)),
        out_shape=jax.ShapeDtypeStruct((B, H), jnp.float32),
    )(p0, p1, p2, W1, b1, W2p, b2p)


def kernel(x0_feat, boundaries1, boundaries2, up0_index, up1_index, batch0,
           batch1, batch2, atom_embed, Wu1, bu1, Wu2, bu2, Wf1, bf1, Wf2, bf2,
           Wc, bc, lin1_W, lin1_b, lin2_W, lin2_b):
    i32 = jnp.int32
    featp = jnp.pad(x0_feat.astype(i32), ((0, N0P - N0), (0, 16 - 9)),
                    constant_values=127)
    b1T = tuple(jnp.pad(boundaries1[:, j].astype(i32), (0, N1P - N1))
                for j in range(2))
    b2T = tuple(jnp.pad(boundaries2[:, j].astype(i32), (0, N2P - N2))
                for j in range(3))
    up0s = jnp.pad(up0_index[0].astype(i32), (0, E0P - E0)).reshape(-1, 128)
    up0d = jnp.pad(up0_index[1].astype(i32), (0, E0P - E0),
                   constant_values=N0).reshape(-1, 128)
    up1s = jnp.pad(up1_index[0].astype(i32), (0, E1P - E1)).reshape(-1, 128)
    up1d = jnp.pad(up1_index[1].astype(i32), (0, E1P - E1),
                   constant_values=N1).reshape(-1, 128)
    bat0 = jnp.pad(batch0.astype(i32), (0, N0P - N0),
                   constant_values=B).reshape(-1, 1)
    bat1 = jnp.pad(batch1.astype(i32), (0, N1P - N1),
                   constant_values=B).reshape(-1, 1)
    bat2 = jnp.pad(batch2.astype(i32), (0, N2P - N2),
                   constant_values=B).reshape(-1, 1)

    x0 = _atom_encode(featp, atom_embed)
    x1 = _gsum(x0, b1T, N1P)
    x2 = _gsum(x1, b2T, N2P)
    xs = [x0, x1, x2]

    nlayers = Wu1.shape[0]
    for l in range(nlayers):
        agg0 = _scatter(xs[0].reshape(N0P * 4, 32), up0s, up0d, N0P, 32)
        f2 = _gsum(xs[1], b2T, N2P)
        f1 = _gsum(xs[0], b1T, N1P)
        agg1 = _scatter(xs[1].reshape(N1P * 8, 16), up1s, up1d, N1P, 16)
        new_xs = []
        for d, (aggu, aggf) in enumerate(((agg0, None), (agg1, f1),
                                          (None, f2))):
            W2cat = jnp.concatenate([Wu2[l, d], Wf2[l, d]], axis=0)
            b2cat = (bu2[l, d] + bf2[l, d]).reshape(1, H)
            new_xs.append(_mlp(
                xs[d], aggu, aggf,
                Wu1[l, d], bu1[l, d].reshape(1, H),
                Wf1[l, d], bf1[l, d].reshape(1, H),
                W2cat, b2cat,
                Wc[l, d], bc[l, d].reshape(1, H)))
        xs = new_xs

    p0 = _pool(xs[0], bat0)
    p1 = _pool(xs[1], bat1)
    p2 = _pool(xs[2], bat2)
    W2p = jnp.pad(lin2_W, ((0, 0), (0, H - lin2_W.shape[1])))
    b2p = jnp.pad(lin2_b, (0, H - lin2_b.shape[0])).reshape(1, H)
    out = _final(p0, p1, p2, lin1_W, lin1_b.reshape(3, 1, 2 * H), W2p, b2p)
    return out[:, :lin2_W.shape[1]]


# speedup vs baseline: 1.0737x; 1.0737x over previous
import functools

import jax
import jax.numpy as jnp
from jax import lax
from jax.experimental import pallas as pl
from jax.experimental.pallas import tpu as pltpu
from jax.experimental.pallas import tpu_sc as plsc

H = 128
B = 128
NCORE = 2
NSUB = 16
NW = NCORE * NSUB

N0, N1, N2 = 50000, 100000, 25000
E0, E1 = 200000, 150000
N0P, N1P, N2P = 53248, 102400, 28672
E0P, E1P = 200704, 151552

_MESH = plsc.VectorSubcoreMesh(core_axis_name="c", subcore_axis_name="s")


def _gsum(table, idxs, ndst_p):
    k = len(idxs)
    rows_w = ndst_p // NW
    nfull = rows_w // 512
    tail = (rows_w % 512) // 128

    GK = 2
    units_w = rows_w // 128
    ngrp = (units_w + GK - 1) // GK

    def units(g):
        return range(g * GK, min((g + 1) * GK, units_w))

    @functools.partial(
        pl.kernel,
        out_type=jax.ShapeDtypeStruct((ndst_p, H), jnp.float32),
        mesh=_MESH,
        scratch_types=[
            pltpu.VMEM((2, k, GK, 128), jnp.int32),
            pltpu.VMEM((2, GK, 128, H), jnp.float32),
            pltpu.SemaphoreType.DMA,
            pltpu.SemaphoreType.DMA,
            pltpu.SemaphoreType.DMA,
            pltpu.SemaphoreType.DMA,
        ],
    )
    def kfn(table_h, *rest):
        idx_hs = rest[:k]
        out_h, idx_v, rows_v, semA, semB, sem2, sem3 = rest[k:]
        cid = lax.axis_index("c")
        sid = lax.axis_index("s")
        wid = sid * NCORE + cid
        base_w = wid * rows_w

        def load_idx(g):
            b = g % 2
            return [pltpu.async_copy(
                        idx_hs[j].at[pl.ds(base_w + u * 128, 128)],
                        idx_v.at[b, j, jj], sem2)
                    for j in range(k) for jj, u in enumerate(units(g))]

        def fire_j(g, j):
            b = g % 2
            sem_ = semA if b == 0 else semB
            return [pltpu.async_copy(table_h.at[idx_v.at[b, j, jj]],
                                     rows_v.at[b, jj], sem_, add=(j > 0))
                    for jj, _ in enumerate(units(g))]

        def fire_wb(g):
            b = g % 2
            return [pltpu.async_copy(
                        rows_v.at[b, jj],
                        out_h.at[pl.ds(base_w + u * 128, 128)], sem3)
                    for jj, u in enumerate(units(g))]

        idx_cur = load_idx(0)
        idx_nxt = load_idx(1) if ngrp > 1 else []
        for cp in idx_cur:
            cp.wait()
        ow = fire_j(0, 0)
        wb_prev, wb_cur = [], []
        for g in range(ngrp):
            for cp in wb_prev:
                cp.wait()
            ow_next = []
            if g + 1 < ngrp:
                for cp in idx_nxt:
                    cp.wait()
                ow_next = fire_j(g + 1, 0)
            for cp in ow:
                cp.wait()
            for j in range(1, k):
                ga = fire_j(g, j)
                for cp in ga:
                    cp.wait()
            wb_prev = wb_cur
            wb_cur = fire_wb(g)
            ow = ow_next
            if g + 1 < ngrp:
                idx_nxt = load_idx(g + 2) if g + 2 < ngrp else []
        for cp in wb_prev:
            cp.wait()
        for cp in wb_cur:
            cp.wait()

    return kfn(table, *idxs)


def _scatter(table, src2d, dst2d, ndst_p, Wc):
    C = H // Wc
    rnds = C // NCORE
    ep = src2d.shape[0] * 128
    per_tile = ep // NSUB
    urows_t = per_tile // 128
    GK = 2 if Wc == 64 else 5
    ngrp = (urows_t + GK - 1) // GK
    rows_tile = ndst_p // NSUB
    zr = rows_tile
    while zr * Wc * 2 > 16384:
        zr //= 2
    nz = rows_tile // zr

    def units(g):
        return range(g * GK, min((g + 1) * GK, urows_t))

    @functools.partial(
        pl.kernel,
        out_type=jax.ShapeDtypeStruct((ndst_p, H), jnp.bfloat16),
        mesh=_MESH,
        compiler_params=pltpu.CompilerParams(use_tc_tiling_on_sc=False),
        scratch_types=[
            pltpu.VMEM_SHARED((ndst_p, Wc), jnp.bfloat16),
            pltpu.VMEM((3, GK, 128), jnp.int32),
            pltpu.VMEM((3, GK, 128), jnp.int32),
            pltpu.VMEM((zr, Wc), jnp.bfloat16),
            pltpu.VMEM((2, GK, 128, Wc), jnp.bfloat16),
            pltpu.SemaphoreType.DMA,
            pltpu.SemaphoreType.DMA,
            pltpu.SemaphoreType.DMA,
            pltpu.SemaphoreType.DMA,
            pltpu.SemaphoreType.DMA,
        ],
    )
    def kfn(table_h, src_h, dst_h, out_h, acc, sidx, didx, zbuf, rows,
            semA, semB, semC, semD, sem2):
        cid = lax.axis_index("c")
        sid = lax.axis_index("s")
        ubase = sid * urows_t

        def zb(i, carry):
            for t in range(Wc // 32):
                zbuf[i, pl.ds(t * 32, 32)] = jnp.zeros((32,), jnp.bfloat16)
            return carry
        lax.fori_loop(0, zr, zb, 0)

        for r in range(rnds):
            zcps = [pltpu.async_copy(
                        zbuf, acc.at[pl.ds(sid * rows_tile + i * zr, zr)],
                        sem2)
                    for i in range(nz)]
            for cp in zcps:
                cp.wait()
            plsc.subcore_barrier()

            col0 = (r * NCORE + cid) * Wc
            chunk_t = r * NCORE + cid

            def load_idx(g):
                b = g % 3
                u0 = g * GK
                nk = len(units(g))
                return [pltpu.async_copy(
                            src_h.at[pl.ds(ubase + u0, nk)],
                            sidx.at[b, pl.ds(0, nk)], sem2),
                        pltpu.async_copy(
                            dst_h.at[pl.ds(ubase + u0, nk)],
                            didx.at[b, pl.ds(0, nk)], sem2)]

            def fire_gathers(g):
                b3 = g % 3
                b = g % 2
                sem_ = semA if b == 0 else semB
                for jj, _ in enumerate(units(g)):
                    for t in range(8):
                        v = sidx[b3, jj, pl.ds(t * 16, 16)]
                        sidx[b3, jj, pl.ds(t * 16, 16)] = v * C + chunk_t
                return [pltpu.async_copy(
                            table_h.at[sidx.at[b3, jj]],
                            rows.at[b, jj], sem_)
                        for jj, _ in enumerate(units(g))]

            def fire_scatter(g):
                b3 = g % 3
                b = g % 2
                sem_ = semC if b == 0 else semD
                return [pltpu.async_copy(rows.at[b, jj],
                                         acc.at[didx.at[b3, jj]], sem_,
                                         add=True)
                        for jj, _ in enumerate(units(g))]

            for cp in load_idx(0):
                cp.wait()
            gat = fire_gathers(0)
            idx_nxt = load_idx(1) if ngrp > 1 else []
            sca_prev = []
            for g in range(ngrp):
                for cp in idx_nxt:
                    cp.wait()
                for cp in sca_prev:
                    cp.wait()
                gat_next = fire_gathers(g + 1) if g + 1 < ngrp else []
                idx_nxt = load_idx(g + 2) if g + 2 < ngrp else []
                for cp in gat:
                    cp.wait()
                sca_prev = fire_scatter(g)
                gat = gat_next
            for cp in sca_prev:
                cp.wait()
            plsc.subcore_barrier()

            ob = sid * rows_tile
            pltpu.sync_copy(acc.at[pl.ds(ob, rows_tile)],
                            out_h.at[pl.ds(ob, rows_tile), pl.ds(col0, Wc)])
            plsc.subcore_barrier()

    return kfn(table, src2d, dst2d)


def _atom_encode(featp, emb):
    n0p = featp.shape[0]
    R = 512

    def body(feat_ref, emb_ref, out_ref, outb_ref):
        acc = jnp.zeros((R, H), jnp.float32)
        for c in range(9):
            col = feat_ref[:, c:c + 1]
            oh = (col == lax.broadcasted_iota(jnp.int32, (R, 64), 1)
                  ).astype(jnp.float32)
            acc = acc + jnp.dot(oh, emb_ref[c],
                                preferred_element_type=jnp.float32)
        out_ref[...] = acc
        outb_ref[...] = acc.astype(jnp.bfloat16)

    return pl.pallas_call(
        body,
        grid=(n0p // R,),
        in_specs=[pl.BlockSpec((R, 16), lambda i: (i, 0)),
                  pl.BlockSpec((9, 64, H), lambda i: (0, 0, 0))],
        out_specs=[pl.BlockSpec((R, H), lambda i: (i, 0)),
                   pl.BlockSpec((R, H), lambda i: (i, 0))],
        out_shape=[jax.ShapeDtypeStruct((n0p, H), jnp.float32),
                   jax.ShapeDtypeStruct((n0p, H), jnp.bfloat16)],
    )(featp, emb)


def _mlp(x, aggu, aggf, W1u, b1u, W1f, b1f, W2cat, b2cat, Wcm, bcm,
         need_b=False):
    np_ = x.shape[0]
    R = 512
    has_u = aggu is not None
    has_f = aggf is not None

    def body(*refs):
        i = 0
        x_ref = refs[i]; i += 1
        if has_u:
            aggu_ref = refs[i]; i += 1
        if has_f:
            aggf_ref = refs[i]; i += 1
        (W1u_ref, b1u_ref, W1f_ref, b1f_ref, W2_ref, b2_ref, Wcm_ref,
         bcm_ref) = refs[i:i + 8]
        out_refs = refs[i + 8:]
        xv = x_ref[...]
        if has_u:
            hu = xv + aggu_ref[...].astype(jnp.float32)
        else:
            hu = xv
        if has_f:
            hf = xv + aggf_ref[...]
        else:
            hf = xv
        au = jnp.maximum(jnp.dot(hu, W1u_ref[...],
                                 preferred_element_type=jnp.float32)
                         + b1u_ref[...], 0.0)
        af = jnp.maximum(jnp.dot(hf, W1f_ref[...],
                                 preferred_element_type=jnp.float32)
                         + b1f_ref[...], 0.0)
        cat = jnp.concatenate([au, af], axis=1)
        s = jnp.dot(cat, W2_ref[...], preferred_element_type=jnp.float32) \
            + b2_ref[...]
        res = jnp.maximum(
            jnp.dot(s, Wcm_ref[...], preferred_element_type=jnp.float32)
            + bcm_ref[...], 0.0)
        out_refs[0][...] = res
        if need_b:
            out_refs[1][...] = res.astype(jnp.bfloat16)

    in_specs = [pl.BlockSpec((R, H), lambda i: (i, 0))]
    args = [x]
    if has_u:
        in_specs.append(pl.BlockSpec((R, H), lambda i: (i, 0)))
        args.append(aggu)
    if has_f:
        in_specs.append(pl.BlockSpec((R, H), lambda i: (i, 0)))
        args.append(aggf)
    wspecs = [
        pl.BlockSpec((H, H), lambda i: (0, 0)),
        pl.BlockSpec((1, H), lambda i: (0, 0)),
        pl.BlockSpec((H, H), lambda i: (0, 0)),
        pl.BlockSpec((1, H), lambda i: (0, 0)),
        pl.BlockSpec((2 * H, H), lambda i: (0, 0)),
        pl.BlockSpec((1, H), lambda i: (0, 0)),
        pl.BlockSpec((H, H), lambda i: (0, 0)),
        pl.BlockSpec((1, H), lambda i: (0, 0)),
    ]
    out_specs = [pl.BlockSpec((R, H), lambda i: (i, 0))]
    out_shape = [jax.ShapeDtypeStruct((np_, H), jnp.float32)]
    if need_b:
        out_specs.append(pl.BlockSpec((R, H), lambda i: (i, 0)))
        out_shape.append(jax.ShapeDtypeStruct((np_, H), jnp.bfloat16))
    res = pl.pallas_call(
        body,
        grid=(np_ // R,),
        in_specs=in_specs + wspecs,
        out_specs=out_specs,
        out_shape=out_shape,
    )(*args, W1u, b1u, W1f, b1f, W2cat, b2cat, Wcm, bcm)
    return res if need_b else res[0]


def _cast16(x):
    np_ = x.shape[0]
    R = 1024

    def body(x_ref, o_ref):
        o_ref[...] = x_ref[...].astype(jnp.bfloat16)

    return pl.pallas_call(
        body,
        grid=(np_ // R,),
        in_specs=[pl.BlockSpec((R, H), lambda i: (i, 0))],
        out_specs=pl.BlockSpec((R, H), lambda i: (i, 0)),
        out_shape=jax.ShapeDtypeStruct((np_, H), jnp.bfloat16),
    )(x)


def _pool(x, batch2d):
    np_ = x.shape[0]
    R = 512

    def body(x_ref, b_ref, out_ref):
        i = pl.program_id(0)
        oh = (b_ref[...] == lax.broadcasted_iota(jnp.int32, (R, B), 1)
              ).astype(jnp.float32)
        p = lax.dot_general(oh, x_ref[...], (((0,), (0,)), ((), ())),
                            preferred_element_type=jnp.float32)

        @pl.when(i == 0)
        def _():
            out_ref[...] = p

        @pl.when(i > 0)
        def _():
            out_ref[...] += p

    return pl.pallas_call(
        body,
        grid=(np_ // R,),
        in_specs=[pl.BlockSpec((R, H), lambda i: (i, 0)),
                  pl.BlockSpec((R, 1), lambda i: (i, 0))],
        out_specs=pl.BlockSpec((B, H), lambda i: (0, 0)),
        out_shape=jax.ShapeDtypeStruct((B, H), jnp.float32),
    )(x, batch2d)


def _final(p0, p1, p2, W1, b1, W2p, b2p):
    def body(p0_ref, p1_ref, p2_ref, W1_ref, b1_ref, W2_ref, b2_ref, out_ref):
        h = jnp.zeros((B, 2 * H), jnp.float32)
        for d, pr in enumerate((p0_ref, p1_ref, p2_ref)):
            h = h + jnp.maximum(
                jnp.dot(pr[...], W1_ref[d], preferred_element_type=jnp.float32)
                + b1_ref[d], 0.0)
        out_ref[...] = jnp.dot(h, W2_ref[...],
                               preferred_element_type=jnp.float32) + b2_ref[...]

    return pl.pallas_call(
        body,
        in_specs=[pl.BlockSpec((B, H), lambda: (0, 0)),
                  pl.BlockSpec((B, H), lambda: (0, 0)),
                  pl.BlockSpec((B, H), lambda: (0, 0)),
                  pl.BlockSpec((3, H, 2 * H), lambda: (0, 0, 0)),
                  pl.BlockSpec((3, 1, 2 * H), lambda: (0, 0, 0)),
                  pl.BlockSpec((2 * H, H), lambda: (0, 0)),
                  pl.BlockSpec((1, H), lambda: (0, 0))],
        out_specs=pl.BlockSpec((B, H), lambda: (0, 0)),
        out_shape=jax.ShapeDtypeStruct((B, H), jnp.float32),
    )(p0, p1, p2, W1, b1, W2p, b2p)


def kernel(x0_feat, boundaries1, boundaries2, up0_index, up1_index, batch0,
           batch1, batch2, atom_embed, Wu1, bu1, Wu2, bu2, Wf1, bf1, Wf2, bf2,
           Wc, bc, lin1_W, lin1_b, lin2_W, lin2_b):
    i32 = jnp.int32
    featp = jnp.pad(x0_feat.astype(i32), ((0, N0P - N0), (0, 16 - 9)),
                    constant_values=127)
    b1T = tuple(jnp.pad(boundaries1[:, j].astype(i32), (0, N1P - N1))
                for j in range(2))
    b2T = tuple(jnp.pad(boundaries2[:, j].astype(i32), (0, N2P - N2))
                for j in range(3))
    up0s = jnp.pad(up0_index[0].astype(i32), (0, E0P - E0)).reshape(-1, 128)
    up0d = jnp.pad(up0_index[1].astype(i32), (0, E0P - E0),
                   constant_values=N0).reshape(-1, 128)
    up1s = jnp.pad(up1_index[0].astype(i32), (0, E1P - E1)).reshape(-1, 128)
    up1d = jnp.pad(up1_index[1].astype(i32), (0, E1P - E1),
                   constant_values=N1).reshape(-1, 128)
    bat0 = jnp.pad(batch0.astype(i32), (0, N0P - N0),
                   constant_values=B).reshape(-1, 1)
    bat1 = jnp.pad(batch1.astype(i32), (0, N1P - N1),
                   constant_values=B).reshape(-1, 1)
    bat2 = jnp.pad(batch2.astype(i32), (0, N2P - N2),
                   constant_values=B).reshape(-1, 1)

    x0, x0b = _atom_encode(featp, atom_embed)
    x1 = _gsum(x0, b1T, N1P)
    x1b = _cast16(x1)
    x2 = _gsum(x1, b2T, N2P)
    xs = [x0, x1, x2]
    xb = [x0b, x1b]

    nlayers = Wu1.shape[0]
    for l in range(nlayers):
        last = l == nlayers - 1
        agg0 = _scatter(xb[0].reshape(N0P * 2, 64), up0s, up0d, N0P, 64)
        f2 = _gsum(xs[1], b2T, N2P)
        f1 = _gsum(xs[0], b1T, N1P)
        agg1 = _scatter(xb[1].reshape(N1P * 4, 32), up1s, up1d, N1P, 32)
        new_xs = []
        new_xb = []
        for d, (aggu, aggf) in enumerate(((agg0, None), (agg1, f1),
                                          (None, f2))):
            W2cat = jnp.concatenate([Wu2[l, d], Wf2[l, d]], axis=0)
            b2cat = (bu2[l, d] + bf2[l, d]).reshape(1, H)
            need_b = (not last) and d < 2
            r = _mlp(
                xs[d], aggu, aggf,
                Wu1[l, d], bu1[l, d].reshape(1, H),
                Wf1[l, d], bf1[l, d].reshape(1, H),
                W2cat, b2cat,
                Wc[l, d], bc[l, d].reshape(1, H), need_b=need_b)
            if need_b:
                new_xs.append(r[0])
                new_xb.append(r[1])
            else:
                new_xs.append(r)
        xs = new_xs
        xb = new_xb

    p0 = _pool(xs[0], bat0)
    p1 = _pool(xs[1], bat1)
    p2 = _pool(xs[2], bat2)
    W2p = jnp.pad(lin2_W, ((0, 0), (0, H - lin2_W.shape[1])))
    b2p = jnp.pad(lin2_b, (0, H - lin2_b.shape[0])).reshape(1, H)
    out = _final(p0, p1, p2, lin1_W, lin1_b.reshape(3, 1, 2 * H), W2p, b2p)
    return out[:, :lin2_W.shape[1]]
